# Initial kernel scaffold; baseline (speedup 1.0000x reference)
import jax
import jax.numpy as jnp
from jax.experimental import pallas as pl


def kernel(x, tok_table, pos_table, gamma, beta):
    raise NotImplementedError("write your pallas kernel here")


# speedup vs baseline: 9.8553x; 9.8553x over previous
import functools

import jax
import jax.numpy as jnp
import numpy as np
from jax import lax
from jax.experimental import pallas as pl
from jax.experimental.pallas import tpu as pltpu
from jax.experimental.pallas import tpu_sc as plsc

VOCAB = 26
SEQ = 200
DM = 64
EPS = 1e-5
ROWS = 16384
TOTAL = ROWS * SEQ
NPAIR = TOTAL // 2
JP = SEQ // 2
COMBO = VOCAB * VOCAB

NC = 2
NS = 16
NW = NC * NS

CHUNK = 128
CPW = NPAIR // (NW * CHUNK)
GRP = 8
BR = 1024

_W = np.zeros((SEQ, JP), np.float32)
for _k in range(JP):
    _W[2 * _k, _k] = float(VOCAB)
    _W[2 * _k + 1, _k] = 1.0
_W.setflags(write=False)


def _fused_body(tok_ref, pos_ref, g_ref, b_ref, tab_ref):
    emb = pos_ref[:SEQ][:, None, :] + tok_ref[...][None, :, :]
    mean = jnp.mean(emb, axis=-1, keepdims=True)
    cent = emb - mean
    var = jnp.mean(cent * cent, axis=-1, keepdims=True)
    normed = cent * lax.rsqrt(var + EPS)
    tab_ref[...] = normed * g_ref[...][None] + b_ref[...][None]


def _fused_table(tok, pos, gamma, beta):
    return pl.pallas_call(
        _fused_body,
        out_shape=jax.ShapeDtypeStruct((SEQ, VOCAB, DM), jnp.float32),
    )(tok, pos, gamma, beta)


def _pair_body(f_ref, p_ref):
    left = f_ref[0][:, None, :]
    right = f_ref[1][None, :, :]
    left = jnp.broadcast_to(left, (VOCAB, VOCAB, DM))
    right = jnp.broadcast_to(right, (VOCAB, VOCAB, DM))
    p_ref[...] = jnp.concatenate([left, right], axis=-1)[None]


def _pair_table(fused):
    return pl.pallas_call(
        _pair_body,
        grid=(JP,),
        in_specs=[pl.BlockSpec((2, VOCAB, DM), lambda jp: (jp, 0, 0))],
        out_specs=pl.BlockSpec((1, VOCAB, VOCAB, 2 * DM), lambda jp: (jp, 0, 0, 0)),
        out_shape=jax.ShapeDtypeStruct((JP, VOCAB, VOCAB, 2 * DM), jnp.float32),
    )(fused)


def _idx_body(x_ref, w_ref, idx_ref):
    xf = x_ref[...].astype(jnp.float32)
    packed = jnp.dot(xf, w_ref[...], preferred_element_type=jnp.float32)
    jpos = lax.broadcasted_iota(jnp.int32, (BR, JP), 1)
    idx_ref[...] = packed.astype(jnp.int32) + jpos * COMBO


def _pair_idx(x, w):
    return pl.pallas_call(
        _idx_body,
        grid=(ROWS // BR,),
        in_specs=[
            pl.BlockSpec((BR, SEQ), lambda i: (i, 0)),
            pl.BlockSpec((SEQ, JP), lambda i: (0, 0)),
        ],
        out_specs=pl.BlockSpec((BR, JP), lambda i: (i, 0)),
        out_shape=jax.ShapeDtypeStruct((ROWS, JP), jnp.int32),
    )(x, w)


_MESH = plsc.VectorSubcoreMesh(
    core_axis_name="c", subcore_axis_name="s", num_cores=NC, num_subcores=NS
)


@functools.partial(
    pl.kernel,
    out_type=jax.ShapeDtypeStruct((NPAIR, 2 * DM), jnp.float32),
    mesh=_MESH,
    scratch_types=[
        pltpu.VMEM((GRP, CHUNK), jnp.int32),
        pltpu.VMEM((CHUNK, 2 * DM), jnp.float32),
        pltpu.VMEM((CHUNK, 2 * DM), jnp.float32),
        pltpu.SemaphoreType.DMA,
        pltpu.SemaphoreType.DMA,
    ],
)
def _gather(idx_hbm, tab_hbm, out_hbm, idxv, rows0, rows1, sem0, sem1):
    wid = lax.axis_index("s") * NC + lax.axis_index("c")
    base = wid * CPW

    bufs = [(rows0, sem0), (rows1, sem1)]

    def group(g, carry):
        row0 = base + g * GRP
        pltpu.sync_copy(idx_hbm.at[pl.ds(row0, GRP)], idxv)
        handles = [None] * GRP
        handles[0] = pltpu.async_copy(tab_hbm.at[idxv.at[0]], rows0, sem0)
        for s in range(GRP):
            cur_rows, _ = bufs[s % 2]
            if s + 1 < GRP:
                nxt_rows, nxt_sem = bufs[(s + 1) % 2]
                handles[s + 1] = pltpu.async_copy(
                    tab_hbm.at[idxv.at[s + 1]], nxt_rows, nxt_sem
                )
            handles[s].wait()
            pltpu.sync_copy(cur_rows, out_hbm.at[pl.ds((row0 + s) * CHUNK, CHUNK)])
        return carry

    lax.fori_loop(0, CPW // GRP, group, 0)


def kernel(x, tok_table, pos_table, gamma, beta):
    x = x.astype(jnp.int32)
    fused = _fused_table(
        tok_table, pos_table, gamma.reshape(1, DM), beta.reshape(1, DM)
    )
    ptab = _pair_table(fused)
    pidx = _pair_idx(x, jnp.asarray(_W))
    out2 = _gather(
        pidx.reshape(NPAIR // CHUNK, CHUNK),
        ptab.reshape(JP * COMBO, 2 * DM),
    )
    return out2.reshape(ROWS, SEQ, DM)


# speedup vs baseline: 38.4922x; 3.9057x over previous
import jax
import jax.numpy as jnp
import numpy as np
from jax import lax
from jax.experimental import pallas as pl

VOCAB = 26
SEQ = 200
DM = 64
EPS = 1e-5
ROWS = 16384

GJ = 8
NG = SEQ // GJ
KK = GJ * VOCAB
MM = GJ * DM
BI = 512
NB = ROWS // BI

_R = np.zeros((KK, GJ), np.float32)
for _r in range(KK):
    _R[_r, _r // VOCAB] = 1.0
_R.setflags(write=False)


def _fused_body(tok_ref, pos_ref, g_ref, b_ref, tab_ref):
    emb = pos_ref[:SEQ][:, None, :] + tok_ref[...][None, :, :]
    mean = jnp.mean(emb, axis=-1, keepdims=True)
    cent = emb - mean
    var = jnp.mean(cent * cent, axis=-1, keepdims=True)
    normed = cent * lax.rsqrt(var + EPS)
    tab_ref[...] = normed * g_ref[...][None] + b_ref[...][None]


def _fused_table(tok, pos, gamma, beta):
    return pl.pallas_call(
        _fused_body,
        out_shape=jax.ShapeDtypeStruct((SEQ, VOCAB, DM), jnp.float32),
    )(tok, pos, gamma, beta)


def _wblk_body(f_ref, w_ref):
    rows = []
    for a in range(GJ):
        wa = jnp.transpose(f_ref[a], (1, 0))
        parts = []
        if a:
            parts.append(jnp.zeros((DM, VOCAB * a), jnp.float32))
        parts.append(wa)
        if a + 1 < GJ:
            parts.append(jnp.zeros((DM, VOCAB * (GJ - 1 - a)), jnp.float32))
        rows.append(jnp.concatenate(parts, axis=1) if len(parts) > 1 else parts[0])
    w_ref[...] = jnp.concatenate(rows, axis=0)[None]


def _wblk(fused):
    return pl.pallas_call(
        _wblk_body,
        grid=(NG,),
        in_specs=[pl.BlockSpec((GJ, VOCAB, DM), lambda g: (g, 0, 0))],
        out_specs=pl.BlockSpec((1, MM, KK), lambda g: (g, 0, 0)),
        out_shape=jax.ShapeDtypeStruct((NG, MM, KK), jnp.float32),
    )(fused)


def _main_body(xt_ref, w_ref, r_ref, out_ref):
    xf = xt_ref[...].reshape(GJ, BI).astype(jnp.float32)
    xrep = jnp.dot(r_ref[...], xf, preferred_element_type=jnp.float32)
    rio = lax.broadcasted_iota(jnp.int32, (KK, BI), 0)
    tpat = (rio - (rio // VOCAB) * VOCAB).astype(jnp.float32)
    oh = (xrep == tpat).astype(jnp.float32)
    res = jnp.dot(w_ref[0], oh, preferred_element_type=jnp.float32)
    out_ref[...] = res.reshape(GJ, DM, BI)


def _main(xt3, wbig, r):
    return pl.pallas_call(
        _main_body,
        grid=(NG, NB),
        in_specs=[
            pl.BlockSpec((GJ, 1, BI), lambda g, b: (g, 0, b)),
            pl.BlockSpec((1, MM, KK), lambda g, b: (g, 0, 0)),
            pl.BlockSpec((KK, GJ), lambda g, b: (0, 0)),
        ],
        out_specs=pl.BlockSpec((GJ, DM, BI), lambda g, b: (g, 0, b)),
        out_shape=jax.ShapeDtypeStruct((SEQ, DM, ROWS), jnp.float32),
    )(xt3, wbig, r)


def kernel(x, tok_table, pos_table, gamma, beta):
    x = x.astype(jnp.int32)
    fused = _fused_table(
        tok_table, pos_table, gamma.reshape(1, DM), beta.reshape(1, DM)
    )
    wbig = _wblk(fused)
    xt3 = jnp.transpose(x).reshape(SEQ, 1, ROWS)
    out_t = _main(xt3, wbig, jnp.asarray(_R))
    return jnp.transpose(out_t, (2, 0, 1))
